# baseline (speedup 1.0000x reference)
import functools

import jax
import jax.numpy as jnp
from jax import lax
from jax.experimental import pallas as pl
from jax.experimental.pallas import tpu as pltpu
from jax.experimental.pallas import tpu_sc as plsc

NUM_NODES = 1000000
EMBED_DIM = 32
BATCH = 16384
_ROWS_PER_128 = 128 // EMBED_DIM

_info = plsc.get_sparse_core_info()
_NC = _info.num_cores
_NS = _info.num_subcores
_NW = _NC * _NS
_B_PER_W = BATCH // _NW
_L = 16
_CHUNK = 128
_NCHUNK = _B_PER_W // _CHUNK
_CGROUPS = _CHUNK // _L

_mesh = plsc.VectorSubcoreMesh(core_axis_name="c", subcore_axis_name="s")


@functools.partial(
    pl.kernel,
    mesh=_mesh,
    out_type=jax.ShapeDtypeStruct((BATCH, EMBED_DIM), jnp.float32),
    scratch_types=[
        pltpu.VMEM((_NCHUNK, _CHUNK), jnp.int32),
        pltpu.VMEM((_B_PER_W,), jnp.int32),
        pltpu.VMEM((_CHUNK, 128), jnp.float32),
        pltpu.VMEM((_B_PER_W, EMBED_DIM), jnp.float32),
        pltpu.SemaphoreType.DMA,
    ],
    compiler_params=pltpu.CompilerParams(needs_layout_passes=False),
)
def _gather_kernel(idx4_hbm, coff_hbm, table_hbm, out_hbm,
                   idx4_v, coff_v, rows_v, out_v, sem):
    wid = lax.axis_index("s") * _NC + lax.axis_index("c")
    base = wid * _B_PER_W
    pltpu.sync_copy(idx4_hbm.at[pl.ds(wid * _NCHUNK, _NCHUNK)], idx4_v)
    pltpu.sync_copy(coff_hbm.at[pl.ds(base, _B_PER_W)], coff_v)

    lanes = lax.iota(jnp.int32, _L)

    for k in range(_NCHUNK):
        pltpu.async_copy(table_hbm.at[idx4_v.at[k]], rows_v, sem).wait()

        def body(g, carry):
            row16 = g * _L + lanes
            c16 = coff_v[pl.ds(k * _CHUNK + g * _L, _L)]
            for j in range(EMBED_DIM):
                val = plsc.load_gather(rows_v, [row16, c16 + j])
                plsc.store_scatter(out_v, [k * _CHUNK + row16, lanes * 0 + j],
                                   val)
            return carry

        lax.fori_loop(0, _CGROUPS, body, 0)

    pltpu.sync_copy(out_v, out_hbm.at[pl.ds(base, _B_PER_W)])


def kernel(node_indices, embeddings):
    idx = node_indices.astype(jnp.int32)
    idx4 = (idx // _ROWS_PER_128).reshape(_NW * _NCHUNK, _CHUNK)
    coff = (idx % _ROWS_PER_128) * EMBED_DIM
    table128 = embeddings.reshape(NUM_NODES // _ROWS_PER_128, 128)
    return _gather_kernel(idx4, coff, table128)


# speedup vs baseline: 1.0402x; 1.0402x over previous
import functools

import jax
import jax.numpy as jnp
from jax import lax
from jax.experimental import pallas as pl
from jax.experimental.pallas import tpu as pltpu
from jax.experimental.pallas import tpu_sc as plsc

NUM_NODES = 1000000
EMBED_DIM = 32
BATCH = 16384

_info = plsc.get_sparse_core_info()
_NC = _info.num_cores
_NS = _info.num_subcores
_NW = _NC * _NS
_B_PER_W = BATCH // _NW

_mesh = plsc.VectorSubcoreMesh(core_axis_name="c", subcore_axis_name="s")


@functools.partial(
    pl.kernel,
    mesh=_mesh,
    out_type=jax.ShapeDtypeStruct((BATCH, EMBED_DIM), jnp.float32),
    scratch_types=[
        pltpu.VMEM((_B_PER_W,), jnp.int32),
        pltpu.VMEM((_B_PER_W, EMBED_DIM), jnp.float32),
        pltpu.SemaphoreType.DMA,
    ],
    compiler_params=pltpu.CompilerParams(use_tc_tiling_on_sc=False),
)
def _gather_kernel(idx_hbm, table_hbm, out_hbm, idx_v, rows_v, sem):
    wid = lax.axis_index("s") * _NC + lax.axis_index("c")
    base = wid * _B_PER_W
    pltpu.sync_copy(idx_hbm.at[pl.ds(base, _B_PER_W)], idx_v)
    pltpu.async_copy(table_hbm.at[idx_v], rows_v, sem).wait()
    pltpu.sync_copy(rows_v, out_hbm.at[pl.ds(base, _B_PER_W)])


def kernel(node_indices, embeddings):
    return _gather_kernel(node_indices.astype(jnp.int32), embeddings)
